# baseline (speedup 1.0000x reference)
import functools

import jax
import jax.numpy as jnp
from jax.experimental import pallas as pl


def _make_kernel(w, k, fac, bh, c):
    def body(f_ref, a_ref, cent_ref, out_ref):
        f2d = f_ref[0].reshape(c * bh, w)
        a = a_ref[...]
        cross = jnp.dot(a, f2d)
        x3 = cross.reshape(k, bh, w)
        cent = cent_ref[...]
        c2 = jnp.sum(cent * cent, axis=1)[:, None, None]
        score = c2 - 2.0 * x3
        smin = jnp.min(score, axis=0, keepdims=True)
        kid = jax.lax.broadcasted_iota(jnp.int32, (k, bh, w), 0)
        m = jnp.min(jnp.where(score == smin, kid, k), axis=0)
        mf = m.astype(jnp.float32)
        col = jax.lax.broadcasted_iota(jnp.int32, (w, w * fac), 1)
        row = jax.lax.broadcasted_iota(jnp.int32, (w, w * fac), 0)
        sel = (col // fac == row).astype(jnp.float32)
        rep = jnp.dot(mf, sel).astype(jnp.int32)
        rep3 = jnp.broadcast_to(rep[:, None, :], (bh, fac, w * fac))
        out_ref[0] = rep3.reshape(bh * fac, w * fac)
    return body


def _assign_and_upsample(feature, centroid, H, W):
    b, c, h, w = feature.shape
    k = centroid.shape[0]
    fac = H // h
    assert H == h * fac and W == w * fac
    bh = 8
    eye = jnp.eye(bh, dtype=centroid.dtype)
    a = (centroid[:, None, :, None] * eye[None, :, None, :]).reshape(k * bh, c * bh)
    return pl.pallas_call(
        _make_kernel(w, k, fac, bh, c),
        grid=(b, h // bh),
        in_specs=[
            pl.BlockSpec((1, c, bh, w), lambda i, j: (i, 0, j, 0)),
            pl.BlockSpec((k * bh, c * bh), lambda i, j: (0, 0)),
            pl.BlockSpec((k, c), lambda i, j: (0, 0)),
        ],
        out_specs=pl.BlockSpec((1, bh * fac, w * fac), lambda i, j: (i, j, 0)),
        out_shape=jax.ShapeDtypeStruct((b, H, W), jnp.int32),
    )(feature, a, centroid)


def kernel(feature_s2t, feature_target, seg_s2t, seg_target, centroid_convert, centroid_target):
    H1, W1 = seg_s2t.shape[1], seg_s2t.shape[2]
    H2, W2 = seg_target.shape[1], seg_target.shape[2]
    mask_s2t_target = _assign_and_upsample(feature_s2t, centroid_target, H1, W1)
    mask_target_s2t = _assign_and_upsample(feature_target, centroid_convert, H2, W2)
    return (mask_s2t_target, mask_target_s2t)


# speedup vs baseline: 1.6410x; 1.6410x over previous
import jax
import jax.numpy as jnp
from jax.experimental import pallas as pl
from jax.experimental.pallas import tpu as pltpu


def _make_kernel(w, k, fac, bh, c):
    def body(f_ref, cent_ref, out_ref, a_ref, c2_ref, sel_ref):
        i = pl.program_id(0)
        j = pl.program_id(1)

        @pl.when(jnp.logical_and(i == 0, j == 0))
        def _build_constants():
            cent = cent_ref[...]
            crep = jnp.broadcast_to(cent[:, None, :], (k, bh, c))
            crep = crep.reshape(k * bh, c)
            colc = jax.lax.broadcasted_iota(jnp.int32, (c, c * bh), 1)
            rowc = jax.lax.broadcasted_iota(jnp.int32, (c, c * bh), 0)
            selc = (colc // bh == rowc).astype(jnp.float32)
            tmp = jnp.dot(crep, selc)
            cola = jax.lax.broadcasted_iota(jnp.int32, (k * bh, c * bh), 1)
            rowa = jax.lax.broadcasted_iota(jnp.int32, (k * bh, c * bh), 0)
            a = jnp.where((cola & (bh - 1)) == (rowa & (bh - 1)), tmp, 0.0)
            a_ref[...] = a
            c2_ref[...] = jnp.sum(a * a, axis=1, keepdims=True)
            colu = jax.lax.broadcasted_iota(jnp.int32, (w, w * fac), 1)
            rowu = jax.lax.broadcasted_iota(jnp.int32, (w, w * fac), 0)
            sel_ref[...] = (colu // fac == rowu).astype(jnp.float32)

        f2d = f_ref[0].reshape(c * bh, w)
        cross = jnp.dot(a_ref[...], f2d)
        score = c2_ref[...] - 2.0 * cross
        s3 = score.reshape(k, bh, w)
        smin = jnp.min(s3, axis=0, keepdims=True)
        kid = jax.lax.broadcasted_iota(jnp.int32, (k, bh, w), 0)
        m = jnp.min(jnp.where(s3 == smin, kid, k), axis=0)
        mf = m.astype(jnp.float32)
        rep = jnp.dot(mf, sel_ref[...]).astype(jnp.int32)
        rep3 = jnp.broadcast_to(rep[:, None, :], (bh, fac, w * fac))
        out_ref[0] = rep3.reshape(bh * fac, w * fac)
    return body


def _assign_and_upsample(feature, centroid, H, W):
    b, c, h, w = feature.shape
    k = centroid.shape[0]
    fac = H // h
    assert H == h * fac and W == w * fac
    bh = 8
    return pl.pallas_call(
        _make_kernel(w, k, fac, bh, c),
        grid=(b, h // bh),
        in_specs=[
            pl.BlockSpec((1, c, bh, w), lambda i, j: (i, 0, j, 0)),
            pl.BlockSpec((k, c), lambda i, j: (0, 0)),
        ],
        out_specs=pl.BlockSpec((1, bh * fac, w * fac), lambda i, j: (i, j, 0)),
        out_shape=jax.ShapeDtypeStruct((b, H, W), jnp.int32),
        scratch_shapes=[
            pltpu.VMEM((k * bh, c * bh), jnp.float32),
            pltpu.VMEM((k * bh, 1), jnp.float32),
            pltpu.VMEM((w, w * fac), jnp.float32),
        ],
    )(feature, centroid)


def kernel(feature_s2t, feature_target, seg_s2t, seg_target, centroid_convert, centroid_target):
    H1, W1 = seg_s2t.shape[1], seg_s2t.shape[2]
    H2, W2 = seg_target.shape[1], seg_target.shape[2]
    mask_s2t_target = _assign_and_upsample(feature_s2t, centroid_target, H1, W1)
    mask_target_s2t = _assign_and_upsample(feature_target, centroid_convert, H2, W2)
    return (mask_s2t_target, mask_target_s2t)
